# baseline (speedup 1.0000x reference)
import functools
import jax
import jax.numpy as jnp
from jax import lax
from jax.experimental import pallas as pl
from jax.experimental.pallas import tpu as pltpu
from jax.experimental.pallas import tpu_sc as plsc

B, L, D = 4096, 200, 128
NC, NS = 2, 16
NW = NC * NS
BPW = B // NW
LA, LB = 128, 72


def _body(x_hbm, tab_hbm, out_hbm, idx_v, rowsA, rowsB, trA, trB,
          gA, gB, oA, oB):
    wid = lax.axis_index("s") * NC + lax.axis_index("c")
    lane = lax.iota(jnp.int32, 16)
    row0 = pl.multiple_of(wid * BPW, 8)

    xoff = pl.multiple_of(row0 * L, 8)
    pltpu.sync_copy(x_hbm.at[pl.ds(xoff, BPW * L)], idx_v)

    def fire_gather(i, loff, lc, dst, sem):
        off = pl.multiple_of(i * L + loff, 8)
        pltpu.async_copy(tab_hbm.at[idx_v.at[pl.ds(off, lc)]],
                         dst.at[pl.ds(0, lc)], sem)

    def wait_gather(lc, dst, sem):
        pltpu.make_async_copy(tab_hbm.at[idx_v.at[pl.ds(0, lc)]],
                              dst.at[pl.ds(0, lc)], sem).wait()

    def transpose(lc, rows_s, tr_s):
        def tr(l, c):
            l_vec = jnp.full((16,), l, jnp.int32)
            for j in range(D // 16):
                v = rows_s[l, pl.ds(j * 16, 16)]
                plsc.store_scatter(tr_s, [j * 16 + lane, l_vec], v)
            return c
        lax.fori_loop(0, lc, tr, 0)

    def out_ref(i, loff, lc):
        return out_hbm.at[row0 + i, :, pl.ds(loff, lc)]

    fire_gather(0, 0, LA, rowsA, gA)
    fire_gather(0, LA, LB, rowsB, gB)

    def step(i, carry):
        for loff, lc, rows_s, tr_s, gs, os in (
                (0, LA, rowsA, trA, gA, oA),
                (LA, LB, rowsB, trB, gB, oB)):
            wait_gather(lc, rows_s, gs)

            @pl.when(i > 0)
            def _():
                pltpu.make_async_copy(tr_s, out_ref(0, loff, lc), os).wait()

            transpose(lc, rows_s, tr_s)

            @pl.when(i < BPW - 1)
            def _():
                fire_gather(i + 1, loff, lc, rows_s, gs)

            pltpu.async_copy(tr_s, out_ref(i, loff, lc), os)
        return carry

    lax.fori_loop(0, BPW, step, 0)
    pltpu.make_async_copy(trA, out_ref(0, 0, LA), oA).wait()
    pltpu.make_async_copy(trB, out_ref(0, LA, LB), oB).wait()


def kernel(x, table):
    x = x.astype(jnp.int32).reshape(B * L)
    mesh = plsc.VectorSubcoreMesh(core_axis_name="c", subcore_axis_name="s")
    out = pl.kernel(
        _body,
        mesh=mesh,
        out_type=jax.ShapeDtypeStruct((B, D, L), jnp.float32),
        compiler_params=pltpu.CompilerParams(needs_layout_passes=False),
        scratch_types=[
            pltpu.VMEM((BPW * L,), jnp.int32),
            pltpu.VMEM((LA, D), jnp.float32),
            pltpu.VMEM((LB, D), jnp.float32),
            pltpu.VMEM((D, LA), jnp.float32),
            pltpu.VMEM((D, LB), jnp.float32),
            pltpu.SemaphoreType.DMA,
            pltpu.SemaphoreType.DMA,
            pltpu.SemaphoreType.DMA,
            pltpu.SemaphoreType.DMA,
        ],
    )(x, table)
    return out


# speedup vs baseline: 1.9130x; 1.9130x over previous
import functools
import jax
import jax.numpy as jnp
from jax import lax
from jax.experimental import pallas as pl
from jax.experimental.pallas import tpu as pltpu
from jax.experimental.pallas import tpu_sc as plsc

B, L, D = 4096, 200, 128
NC, NS = 2, 16
NW = NC * NS
BPW = B // NW
C0, C1 = 104, 96
NBUF = 4
TCR = 8


def _gather_body(x_hbm, tab_hbm, out_hbm, idx_v, rows, g, o):
    wid = lax.axis_index("s") * NC + lax.axis_index("c")
    row0 = pl.multiple_of(wid * BPW, 8)

    xoff = pl.multiple_of(row0 * L, 8)
    pltpu.sync_copy(x_hbm.at[pl.ds(xoff, BPW * L)], idx_v)

    def fire_gather(i, dst, sem):
        off = pl.multiple_of(i * L, 8)
        pltpu.async_copy(tab_hbm.at[idx_v.at[pl.ds(off, C0)]],
                         dst.at[pl.ds(0, C0)], sem)
        pltpu.async_copy(tab_hbm.at[idx_v.at[pl.ds(off + C0, C1)]],
                         dst.at[pl.ds(C0, C1)], sem)

    def wait_gather(dst, sem):
        pltpu.make_async_copy(tab_hbm.at[idx_v.at[pl.ds(0, C0)]],
                              dst.at[pl.ds(0, C0)], sem).wait()
        pltpu.make_async_copy(tab_hbm.at[idx_v.at[pl.ds(0, C1)]],
                              dst.at[pl.ds(C0, C1)], sem).wait()

    def out_ref(i):
        ooff = pl.multiple_of((row0 + i) * L, 8)
        return out_hbm.at[pl.ds(ooff, L)]

    for q in range(NBUF):
        fire_gather(q, rows[q], g[q])

    def step(p, carry):
        for s in range(NBUF):
            j = p * NBUF + s
            wait_gather(rows[s], g[s])
            pltpu.async_copy(rows[s], out_ref(j), o[s])

            @pl.when(p < BPW // NBUF - 1)
            def _():
                pltpu.make_async_copy(rows[s], out_ref(0), o[s]).wait()
                fire_gather(j + NBUF, rows[s], g[s])
        return carry

    lax.fori_loop(0, BPW // NBUF, step, 0)
    for q in range(NBUF):
        pltpu.make_async_copy(rows[q], out_ref(0), o[q]).wait()


def _tr_body(in_ref, out_ref):
    for r in range(TCR):
        out_ref[r] = in_ref[pl.ds(r * L, L), :].T


def kernel(x, table):
    xf = x.astype(jnp.int32).reshape(B * L)
    mesh = plsc.VectorSubcoreMesh(core_axis_name="c", subcore_axis_name="s")
    emb = pl.kernel(
        _gather_body,
        mesh=mesh,
        out_type=jax.ShapeDtypeStruct((B * L, D), jnp.float32),
        compiler_params=pltpu.CompilerParams(needs_layout_passes=False),
        scratch_types=[
            pltpu.VMEM((BPW * L,), jnp.int32),
            [pltpu.VMEM((L, D), jnp.float32) for _ in range(NBUF)],
            [pltpu.SemaphoreType.DMA for _ in range(NBUF)],
            [pltpu.SemaphoreType.DMA for _ in range(NBUF)],
        ],
    )(xf, table)

    out = pl.pallas_call(
        _tr_body,
        grid=(B // TCR,),
        in_specs=[pl.BlockSpec((TCR * L, D), lambda b: (b, 0))],
        out_specs=pl.BlockSpec((TCR, D, L), lambda b: (b, 0, 0)),
        out_shape=jax.ShapeDtypeStruct((B, D, L), jnp.float32),
    )(emb)
    return out


# speedup vs baseline: 2.3046x; 1.2047x over previous
import functools
import jax
import jax.numpy as jnp
from jax import lax
from jax.experimental import pallas as pl
from jax.experimental.pallas import tpu as pltpu
from jax.experimental.pallas import tpu_sc as plsc

B, L, D = 4096, 200, 128
NC, NS = 2, 16
NW = NC * NS
BPW = B // NW
C0, C1 = 104, 96
NBUF = 4
TCR = 32


def _gather_body(x_hbm, tab_hbm, out_hbm, idx_v, rows, g, o):
    wid = lax.axis_index("s") * NC + lax.axis_index("c")
    row0 = pl.multiple_of(wid * BPW, 8)

    xoff = pl.multiple_of(row0 * L, 8)
    pltpu.sync_copy(x_hbm.at[pl.ds(xoff, BPW * L)], idx_v)

    def fire_gather(i, dst, sem):
        off = pl.multiple_of(i * L, 8)
        pltpu.async_copy(tab_hbm.at[idx_v.at[pl.ds(off, C0)]],
                         dst.at[pl.ds(0, C0)], sem)
        pltpu.async_copy(tab_hbm.at[idx_v.at[pl.ds(off + C0, C1)]],
                         dst.at[pl.ds(C0, C1)], sem)

    def wait_gather(dst, sem):
        pltpu.make_async_copy(tab_hbm.at[idx_v.at[pl.ds(0, C0)]],
                              dst.at[pl.ds(0, C0)], sem).wait()
        pltpu.make_async_copy(tab_hbm.at[idx_v.at[pl.ds(0, C1)]],
                              dst.at[pl.ds(C0, C1)], sem).wait()

    def out_ref(i):
        ooff = pl.multiple_of((row0 + i) * L, 8)
        return out_hbm.at[pl.ds(ooff, L)]

    for q in range(NBUF):
        fire_gather(q, rows[q], g[q])

    def step(p, carry):
        for s in range(NBUF):
            j = p * NBUF + s
            wait_gather(rows[s], g[s])
            pltpu.async_copy(rows[s], out_ref(j), o[s])

            @pl.when(p < BPW // NBUF - 1)
            def _():
                pltpu.make_async_copy(rows[s], out_ref(0), o[s]).wait()
                fire_gather(j + NBUF, rows[s], g[s])
        return carry

    lax.fori_loop(0, BPW // NBUF, step, 0)
    for q in range(NBUF):
        pltpu.make_async_copy(rows[q], out_ref(0), o[q]).wait()


def _tr_body(in_ref, out_ref):
    for r in range(TCR):
        out_ref[r] = in_ref[pl.ds(r * L, L), :].T


def kernel(x, table):
    xf = x.astype(jnp.int32).reshape(B * L)
    mesh = plsc.VectorSubcoreMesh(core_axis_name="c", subcore_axis_name="s")
    emb = pl.kernel(
        _gather_body,
        mesh=mesh,
        out_type=jax.ShapeDtypeStruct((B * L, D), jnp.float32),
        compiler_params=pltpu.CompilerParams(needs_layout_passes=False),
        scratch_types=[
            pltpu.VMEM((BPW * L,), jnp.int32),
            [pltpu.VMEM((L, D), jnp.float32) for _ in range(NBUF)],
            [pltpu.SemaphoreType.DMA for _ in range(NBUF)],
            [pltpu.SemaphoreType.DMA for _ in range(NBUF)],
        ],
    )(xf, table)

    out = pl.pallas_call(
        _tr_body,
        grid=(B // TCR,),
        in_specs=[pl.BlockSpec((TCR * L, D), lambda b: (b, 0))],
        out_specs=pl.BlockSpec((TCR, D, L), lambda b: (b, 0, 0)),
        out_shape=jax.ShapeDtypeStruct((B, D, L), jnp.float32),
    )(emb)
    return out


# speedup vs baseline: 2.3264x; 1.0095x over previous
import functools
import jax
import jax.numpy as jnp
from jax import lax
from jax.experimental import pallas as pl
from jax.experimental.pallas import tpu as pltpu
from jax.experimental.pallas import tpu_sc as plsc

B, L, D = 4096, 200, 128
NC, NS = 2, 16
NW = NC * NS
BPW = B // NW
C0, C1 = 104, 96
NBUF = 4
TCR = 64


def _gather_body(x_hbm, tab_hbm, out_hbm, idx_v, rows, g, o):
    wid = lax.axis_index("s") * NC + lax.axis_index("c")
    row0 = pl.multiple_of(wid * BPW, 8)

    xoff = pl.multiple_of(row0 * L, 8)
    pltpu.sync_copy(x_hbm.at[pl.ds(xoff, BPW * L)], idx_v)

    def fire_gather(i, dst, sem):
        off = pl.multiple_of(i * L, 8)
        pltpu.async_copy(tab_hbm.at[idx_v.at[pl.ds(off, C0)]],
                         dst.at[pl.ds(0, C0)], sem)
        pltpu.async_copy(tab_hbm.at[idx_v.at[pl.ds(off + C0, C1)]],
                         dst.at[pl.ds(C0, C1)], sem)

    def wait_gather(dst, sem):
        pltpu.make_async_copy(tab_hbm.at[idx_v.at[pl.ds(0, C0)]],
                              dst.at[pl.ds(0, C0)], sem).wait()
        pltpu.make_async_copy(tab_hbm.at[idx_v.at[pl.ds(0, C1)]],
                              dst.at[pl.ds(C0, C1)], sem).wait()

    def out_ref(i):
        ooff = pl.multiple_of((row0 + i) * L, 8)
        return out_hbm.at[pl.ds(ooff, L)]

    for q in range(NBUF):
        fire_gather(q, rows[q], g[q])

    def step(p, carry):
        for s in range(NBUF):
            j = p * NBUF + s
            wait_gather(rows[s], g[s])
            pltpu.async_copy(rows[s], out_ref(j), o[s])

            @pl.when(p < BPW // NBUF - 1)
            def _():
                pltpu.make_async_copy(rows[s], out_ref(0), o[s]).wait()
                fire_gather(j + NBUF, rows[s], g[s])
        return carry

    lax.fori_loop(0, BPW // NBUF, step, 0)
    for q in range(NBUF):
        pltpu.make_async_copy(rows[q], out_ref(0), o[q]).wait()


def _tr_body(in_ref, out_ref):
    for r in range(TCR):
        out_ref[r] = in_ref[pl.ds(r * L, L), :].T


def kernel(x, table):
    xf = x.astype(jnp.int32).reshape(B * L)
    mesh = plsc.VectorSubcoreMesh(core_axis_name="c", subcore_axis_name="s")
    emb = pl.kernel(
        _gather_body,
        mesh=mesh,
        out_type=jax.ShapeDtypeStruct((B * L, D), jnp.float32),
        compiler_params=pltpu.CompilerParams(needs_layout_passes=False),
        scratch_types=[
            pltpu.VMEM((BPW * L,), jnp.int32),
            [pltpu.VMEM((L, D), jnp.float32) for _ in range(NBUF)],
            [pltpu.SemaphoreType.DMA for _ in range(NBUF)],
            [pltpu.SemaphoreType.DMA for _ in range(NBUF)],
        ],
    )(xf, table)

    out = pl.pallas_call(
        _tr_body,
        grid=(B // TCR,),
        in_specs=[pl.BlockSpec((TCR * L, D), lambda b: (b, 0))],
        out_specs=pl.BlockSpec((TCR, D, L), lambda b: (b, 0, 0)),
        out_shape=jax.ShapeDtypeStruct((B, D, L), jnp.float32),
    )(emb)
    return out


# speedup vs baseline: 2.3439x; 1.0075x over previous
import functools
import jax
import jax.numpy as jnp
from jax import lax
from jax.experimental import pallas as pl
from jax.experimental.pallas import tpu as pltpu
from jax.experimental.pallas import tpu_sc as plsc

B, L, D = 4096, 200, 128
NC, NS = 2, 16
NW = NC * NS
NCH = 4
CB = B // NCH
BPW = CB // NW
C0, C1 = 104, 96
NBUF = 4
TCR = 64


def _gather_body(x_hbm, tab_hbm, out_hbm, idx_v, rows, g, o):
    wid = lax.axis_index("s") * NC + lax.axis_index("c")
    row0 = pl.multiple_of(wid * BPW, 8)

    xoff = pl.multiple_of(row0 * L, 8)
    pltpu.sync_copy(x_hbm.at[pl.ds(xoff, BPW * L)], idx_v)

    def fire_gather(i, dst, sem):
        off = pl.multiple_of(i * L, 8)
        pltpu.async_copy(tab_hbm.at[idx_v.at[pl.ds(off, C0)]],
                         dst.at[pl.ds(0, C0)], sem)
        pltpu.async_copy(tab_hbm.at[idx_v.at[pl.ds(off + C0, C1)]],
                         dst.at[pl.ds(C0, C1)], sem)

    def wait_gather(dst, sem):
        pltpu.make_async_copy(tab_hbm.at[idx_v.at[pl.ds(0, C0)]],
                              dst.at[pl.ds(0, C0)], sem).wait()
        pltpu.make_async_copy(tab_hbm.at[idx_v.at[pl.ds(0, C1)]],
                              dst.at[pl.ds(C0, C1)], sem).wait()

    def out_ref(i):
        ooff = pl.multiple_of((row0 + i) * L, 8)
        return out_hbm.at[pl.ds(ooff, L)]

    for q in range(NBUF):
        fire_gather(q, rows[q], g[q])

    def step(p, carry):
        for s in range(NBUF):
            j = p * NBUF + s
            wait_gather(rows[s], g[s])
            pltpu.async_copy(rows[s], out_ref(j), o[s])

            @pl.when(p < BPW // NBUF - 1)
            def _():
                pltpu.make_async_copy(rows[s], out_ref(0), o[s]).wait()
                fire_gather(j + NBUF, rows[s], g[s])
        return carry

    lax.fori_loop(0, BPW // NBUF, step, 0)
    for q in range(NBUF):
        pltpu.make_async_copy(rows[q], out_ref(0), o[q]).wait()


def _tr_body(in_ref, out_ref):
    for r in range(TCR):
        out_ref[r] = in_ref[pl.ds(r * L, L), :].T


def _tr_body_chained(in_ref, prev_ref, out_ref):
    del prev_ref
    for r in range(TCR):
        out_ref[r] = in_ref[pl.ds(r * L, L), :].T


def kernel(x, table):
    xf = x.astype(jnp.int32).reshape(B * L)
    mesh = plsc.VectorSubcoreMesh(core_axis_name="c", subcore_axis_name="s")
    sc_gather = pl.kernel(
        _gather_body,
        mesh=mesh,
        out_type=jax.ShapeDtypeStruct((CB * L, D), jnp.float32),
        compiler_params=pltpu.CompilerParams(needs_layout_passes=False),
        scratch_types=[
            pltpu.VMEM((BPW * L,), jnp.int32),
            [pltpu.VMEM((L, D), jnp.float32) for _ in range(NBUF)],
            [pltpu.SemaphoreType.DMA for _ in range(NBUF)],
            [pltpu.SemaphoreType.DMA for _ in range(NBUF)],
        ],
    )
    embs = [sc_gather(xf[c * CB * L:(c + 1) * CB * L], table)
            for c in range(NCH)]

    out_shape = jax.ShapeDtypeStruct((B, D, L), jnp.float32)
    emb_spec = pl.BlockSpec((TCR * L, D), lambda b: (b, 0))

    def out_spec(c):
        return pl.BlockSpec((TCR, D, L),
                            lambda b, c=c: (c * (CB // TCR) + b, 0, 0))

    out = pl.pallas_call(
        _tr_body,
        grid=(CB // TCR,),
        in_specs=[emb_spec],
        out_specs=out_spec(0),
        out_shape=out_shape,
    )(embs[0])
    for c in range(1, NCH):
        out = pl.pallas_call(
            _tr_body_chained,
            grid=(CB // TCR,),
            in_specs=[emb_spec, pl.BlockSpec(memory_space=pl.ANY)],
            out_specs=out_spec(c),
            out_shape=out_shape,
            input_output_aliases={1: 0},
        )(embs[c], out)
    return out
